# baseline (speedup 1.0000x reference)
import functools

import jax
import jax.numpy as jnp
from jax.experimental import pallas as pl
from jax.experimental.pallas import tpu as pltpu


def _lora_tile_kernel(x_ref, slot_ref, a_ref, b_ref, o_ref, a_scr, *,
                      num_modules, num_slots, rank, out_size):
    sr = num_slots * rank
    width = num_modules * sr

    @pl.when(pl.program_id(0) == 0)
    def _stack_a():
        for m in range(num_modules):
            for s in range(num_slots):
                c = m * sr + s * rank
                a_scr[:, c:c + rank] = a_ref[m, s]

    x = x_ref[...]
    slots = slot_ref[0, 0, :]
    tt = x.shape[0]

    inter = jnp.dot(x, a_scr[...], preferred_element_type=jnp.float32)
    col_slot = (jax.lax.broadcasted_iota(jnp.int32, (tt, width), 1) // rank) % num_slots
    inter = jnp.where(col_slot == slots[:, None], inter, 0.0)

    for m in range(num_modules):
        o_ref[:, m * out_size:(m + 1) * out_size] = jnp.dot(
            inter[:, m * sr:(m + 1) * sr], b_ref[m],
            preferred_element_type=jnp.float32)


def kernel(x, lora_a, lora_b, slot_ids, layer_idx):
    del layer_idx
    tokens, d_model = x.shape
    num_modules, num_slots, _, rank = lora_a.shape
    out_size = lora_b.shape[-1]
    sr = num_slots * rank
    width = num_modules * sr

    tile = 256
    grid = tokens // tile

    b_all = lora_b.reshape(num_modules, sr, out_size)
    slot3 = slot_ids.reshape(grid, 1, tile)

    body = functools.partial(_lora_tile_kernel, num_modules=num_modules,
                             num_slots=num_slots, rank=rank, out_size=out_size)

    return pl.pallas_call(
        body,
        grid=(grid,),
        in_specs=[
            pl.BlockSpec((tile, d_model), lambda i: (i, 0)),
            pl.BlockSpec((1, 1, tile), lambda i: (i, 0, 0)),
            pl.BlockSpec((num_modules, num_slots, d_model, rank),
                         lambda i: (0, 0, 0, 0)),
            pl.BlockSpec((num_modules, sr, out_size), lambda i: (0, 0, 0)),
        ],
        out_specs=pl.BlockSpec((tile, num_modules * out_size), lambda i: (i, 0)),
        out_shape=jax.ShapeDtypeStruct((tokens, num_modules * out_size), x.dtype),
        scratch_shapes=[pltpu.VMEM((d_model, width), jnp.float32)],
    )(x, slot3, lora_a, b_all)


# speedup vs baseline: 1.2554x; 1.2554x over previous
import functools

import jax
import jax.numpy as jnp
from jax.experimental import pallas as pl
from jax.experimental.pallas import tpu as pltpu


def _lora_tile_kernel(x_ref, slot_ref, a_ref, b_ref, o_ref, *, num_modules,
                      num_slots, rank, out_size):
    x = x_ref[...]
    slots = slot_ref[0, 0, :]
    tt = x.shape[0]
    sr = num_slots * rank
    width = num_modules * sr

    inter = jnp.dot(x, a_ref[...], preferred_element_type=jnp.float32)
    col_slot = (jax.lax.broadcasted_iota(jnp.int32, (tt, width), 1) // rank) % num_slots
    inter = jnp.where(col_slot == slots[:, None], inter, 0.0)

    for m in range(num_modules):
        o_ref[:, m * out_size:(m + 1) * out_size] = jnp.dot(
            inter[:, m * sr:(m + 1) * sr], b_ref[m],
            preferred_element_type=jnp.float32)


def kernel(x, lora_a, lora_b, slot_ids, layer_idx):
    del layer_idx
    tokens, d_model = x.shape
    num_modules, num_slots, _, rank = lora_a.shape
    out_size = lora_b.shape[-1]
    sr = num_slots * rank
    width = num_modules * sr

    tile = 256
    grid = tokens // tile

    a_all = lora_a.transpose(2, 0, 1, 3).reshape(d_model, width)
    b_all = lora_b.reshape(num_modules, sr, out_size)
    slot3 = slot_ids.reshape(grid, 1, tile)

    body = functools.partial(_lora_tile_kernel, num_modules=num_modules,
                             num_slots=num_slots, rank=rank, out_size=out_size)

    return pl.pallas_call(
        body,
        grid=(grid,),
        in_specs=[
            pl.BlockSpec((tile, d_model), lambda i: (i, 0)),
            pl.BlockSpec((1, 1, tile), lambda i: (i, 0, 0)),
            pl.BlockSpec((d_model, width), lambda i: (0, 0)),
            pl.BlockSpec((num_modules, sr, out_size), lambda i: (0, 0, 0)),
        ],
        out_specs=pl.BlockSpec((tile, num_modules * out_size), lambda i: (i, 0)),
        out_shape=jax.ShapeDtypeStruct((tokens, num_modules * out_size), x.dtype),
        compiler_params=pltpu.CompilerParams(
            dimension_semantics=("parallel",)),
    )(x, slot3, a_all, b_all)
